# baseline (speedup 1.0000x reference)
import functools

import jax
import jax.numpy as jnp
from jax import lax
from jax.experimental import pallas as pl
from jax.experimental.pallas import tpu as pltpu
from jax.experimental.pallas import tpu_sc as plsc


_NSLOT = 4


def _sc_gather(tableT, idx):
    D, V = tableT.shape
    B = idx.shape[0]
    info = plsc.get_sparse_core_info()
    num_workers = info.num_cores * info.num_subcores
    b_per_w = B // num_workers
    mesh = plsc.VectorSubcoreMesh(core_axis_name="c", subcore_axis_name="s")

    @functools.partial(
        pl.kernel,
        mesh=mesh,
        out_type=jax.ShapeDtypeStruct((B, D), jnp.float32),
        scratch_types=[
            pltpu.VMEM((b_per_w + 16,), jnp.int32),
            pltpu.VMEM((b_per_w, D), jnp.float32),
            [pltpu.VMEM((D, 128), jnp.float32) for _ in range(_NSLOT)],
            [pltpu.SemaphoreType.DMA for _ in range(_NSLOT)],
        ],
        compiler_params=pltpu.CompilerParams(
            use_tc_tiling_on_sc=True, needs_layout_passes=False
        ),
    )
    def gather_kernel(table_hbm, idx_hbm, out_hbm, idx_v, rows_v, blocks, sems):
        wid = lax.axis_index("s") * info.num_cores + lax.axis_index("c")
        base = wid * b_per_w
        pltpu.sync_copy(
            idx_hbm.at[pl.ds(base, b_per_w)], idx_v.at[pl.ds(0, b_per_w)]
        )
        lane = lax.iota(jnp.int32, 16)

        def fetch(j, s):
            ii = idx_v[pl.ds(j, 16)][0]
            off = pl.multiple_of((ii >> 7) * 128, 128)
            pltpu.async_copy(
                table_hbm.at[:, pl.ds(off, 128)], blocks[s], sems[s]
            )

        for s in range(_NSLOT):
            fetch(s, s)

        def step(g, _):
            for s in range(_NSLOT):
                j = g * _NSLOT + s
                pltpu.make_async_copy(
                    table_hbm.at[:, pl.ds(0, 128)], blocks[s], sems[s]
                ).wait()
                ii = idx_v[pl.ds(j, 16)][0]
                l16 = jnp.full((16,), ii & 127, jnp.int32)
                for q in range(D // 16):
                    xg = plsc.load_gather(blocks[s], [lane + (q * 16), l16])
                    rows_v[j, pl.ds(q * 16, 16)] = xg

                @pl.when(j + _NSLOT < b_per_w)
                def _():
                    fetch(j + _NSLOT, s)

            return _

        lax.fori_loop(0, b_per_w // _NSLOT, step, None)
        pltpu.sync_copy(rows_v, out_hbm.at[pl.ds(base, b_per_w)])

    return gather_kernel(tableT, idx)


def _tc_mlp(x, W1, b1, W2T, b2):
    B, D = x.shape
    H = W1.shape[1]
    BLK = 2048
    dn = (((1,), (1,)), ((), ()))

    def body(x_ref, w1_ref, b1_ref, w2t_ref, b2_ref, o_ref):
        xb = x_ref[...]
        h = jnp.maximum(
            jnp.dot(xb, w1_ref[...], preferred_element_type=jnp.float32)
            + b1_ref[...],
            0.0,
        )
        y = jnp.maximum(
            xb
            + lax.dot_general(
                h, w2t_ref[...], dn, preferred_element_type=jnp.float32
            )
            + b2_ref[...],
            0.0,
        )
        o_ref[...] = y.T

    return pl.pallas_call(
        body,
        grid=(B // BLK,),
        in_specs=[
            pl.BlockSpec((BLK, D), lambda i: (i, 0)),
            pl.BlockSpec((D, H), lambda i: (0, 0)),
            pl.BlockSpec((1, H), lambda i: (0, 0)),
            pl.BlockSpec((D, H), lambda i: (0, 0)),
            pl.BlockSpec((1, D), lambda i: (0, 0)),
        ],
        out_specs=pl.BlockSpec((D, BLK), lambda i: (0, i)),
        out_shape=jax.ShapeDtypeStruct((D, B), jnp.float32),
    )(x, W1, b1.reshape(1, H), W2T, b2.reshape(1, D))


def kernel(a, table, W1, b1, W2, b2):
    x = _sc_gather(table.T, a.astype(jnp.int32))
    outT = _tc_mlp(x, W1, b1, W2.T, b2)
    return outT.T
